# Initial kernel scaffold; baseline (speedup 1.0000x reference)
import jax
import jax.numpy as jnp
from jax.experimental import pallas as pl


def kernel(subword_indices, emb, W1, b1, W2, b2):
    raise NotImplementedError("write your pallas kernel here")


# speedup vs baseline: 5.2513x; 5.2513x over previous
import functools

import jax
import jax.numpy as jnp
from jax import lax
from jax.experimental import pallas as pl
from jax.experimental.pallas import tpu as pltpu
from jax.experimental.pallas import tpu_sc as plsc

B = 4096
L = 200
EMB_DIM = 64
HIDDEN = 256
OUT = 5

NC, NS = 2, 16
NW = NC * NS
ROWS_PER_W = B // NW
CHUNK = 100
CHUNKS_PER_ROW = L // CHUNK
CHUNKS_PER_W = ROWS_PER_W * CHUNKS_PER_ROW
NJ = EMB_DIM // 16

OUT_PAD = 128


def _pool_body(idx_hbm, emb_hbm, out_hbm, idx_v, buf0, buf1, acc_v, sem0, sem1):
    wid = lax.axis_index("s") * NC + lax.axis_index("c")
    cbase = wid * CHUNKS_PER_W
    pltpu.sync_copy(idx_hbm.at[pl.ds(cbase, CHUNKS_PER_W)], idx_v)

    def row_body(r, carry):
        c0 = 2 * r
        h0 = pltpu.async_copy(emb_hbm.at[idx_v.at[c0]], buf0, sem0)
        h1 = pltpu.async_copy(emb_hbm.at[idx_v.at[c0 + 1]], buf1, sem1)
        h0.wait()
        a = [buf0[0, pl.ds(16 * j, 16)] for j in range(NJ)]
        for t in range(1, CHUNK):
            for j in range(NJ):
                a[j] = a[j] + buf0[t, pl.ds(16 * j, 16)]
        h1.wait()
        for t in range(CHUNK):
            for j in range(NJ):
                a[j] = a[j] + buf1[t, pl.ds(16 * j, 16)]
        for j in range(NJ):
            acc_v[r, pl.ds(16 * j, 16)] = a[j]
        return carry

    lax.fori_loop(0, ROWS_PER_W, row_body, 0)
    pltpu.sync_copy(acc_v, out_hbm.at[pl.ds(wid * ROWS_PER_W, ROWS_PER_W)])


@functools.lru_cache(maxsize=1)
def _make_pool():
    return pl.kernel(
        _pool_body,
        out_type=jax.ShapeDtypeStruct((B, EMB_DIM), jnp.float32),
        mesh=plsc.VectorSubcoreMesh(core_axis_name="c", subcore_axis_name="s"),
        compiler_params=pltpu.CompilerParams(use_tc_tiling_on_sc=False),
        scratch_types=[
            pltpu.VMEM((CHUNKS_PER_W, CHUNK), jnp.int32),
            pltpu.VMEM((CHUNK, EMB_DIM), jnp.float32),
            pltpu.VMEM((CHUNK, EMB_DIM), jnp.float32),
            pltpu.VMEM((ROWS_PER_W, EMB_DIM), jnp.float32),
            pltpu.SemaphoreType.DMA,
            pltpu.SemaphoreType.DMA,
        ],
    )


def _mlp_body(x_ref, w1t_ref, b1_ref, w2t_ref, b2_ref, o_ref):
    x = x_ref[:] * (1.0 / L)
    h = jnp.dot(x, w1t_ref[:], preferred_element_type=jnp.float32) + b1_ref[:]
    h = jnp.maximum(h, 0.0)
    o = jnp.dot(h, w2t_ref[:], preferred_element_type=jnp.float32) + b2_ref[:]
    m = jnp.max(o, axis=1, keepdims=True)
    lse = jnp.log(jnp.sum(jnp.exp(o - m), axis=1, keepdims=True)) + m
    o_ref[:] = o - lse


def _mlp(sums, w1t, b1_2d, w2tp, b2p):
    blk = B // 4
    return pl.pallas_call(
        _mlp_body,
        grid=(4,),
        in_specs=[
            pl.BlockSpec((blk, EMB_DIM), lambda i: (i, 0)),
            pl.BlockSpec((EMB_DIM, HIDDEN), lambda i: (0, 0)),
            pl.BlockSpec((1, HIDDEN), lambda i: (0, 0)),
            pl.BlockSpec((HIDDEN, OUT_PAD), lambda i: (0, 0)),
            pl.BlockSpec((1, OUT_PAD), lambda i: (0, 0)),
        ],
        out_specs=pl.BlockSpec((blk, OUT_PAD), lambda i: (i, 0)),
        out_shape=jax.ShapeDtypeStruct((B, OUT_PAD), jnp.float32),
    )(sums, w1t, b1_2d, w2tp, b2p)


def kernel(subword_indices, emb, W1, b1, W2, b2):
    idx = subword_indices.astype(jnp.int32).reshape(B * L // CHUNK, CHUNK)
    sums = _make_pool()(idx, emb)
    w1t = W1.T
    b1_2d = b1.reshape(1, HIDDEN)
    w2tp = jnp.zeros((HIDDEN, OUT_PAD), jnp.float32).at[:, :OUT].set(W2.T)
    b2p = jnp.full((1, OUT_PAD), -1e30, jnp.float32).at[0, :OUT].set(b2)
    out = _mlp(sums, w1t, b1_2d, w2tp, b2p)
    return out[:, :OUT]


# speedup vs baseline: 8.0218x; 1.5276x over previous
import functools

import jax
import jax.numpy as jnp
from jax import lax
from jax.experimental import pallas as pl
from jax.experimental.pallas import tpu as pltpu
from jax.experimental.pallas import tpu_sc as plsc

B = 4096
L = 200
EMB_DIM = 64
HIDDEN = 256
OUT = 5

NC, NS = 2, 16
NW = NC * NS
ROWS_PER_W = B // NW
CHUNK = 100
CHUNKS_PER_ROW = L // CHUNK
CHUNKS_PER_W = ROWS_PER_W * CHUNKS_PER_ROW
NJ = EMB_DIM // 16

OUT_PAD = 128


NBUF = 4
NGRP = CHUNKS_PER_W // NBUF
HALF = CHUNK // 2


def _pool_body(idx_hbm, emb_hbm, out_hbm, idx_v, b0, b1, b2, b3,
               acc_v, s0, s1, s2, s3):
    bufs = (b0, b1, b2, b3)
    sems = (s0, s1, s2, s3)
    wid = lax.axis_index("s") * NC + lax.axis_index("c")
    cbase = wid * CHUNKS_PER_W
    pltpu.sync_copy(idx_hbm.at[pl.ds(cbase, CHUNKS_PER_W)], idx_v)
    for b in range(NBUF):
        pltpu.async_copy(emb_hbm.at[idx_v.at[b]], bufs[b], sems[b])

    def grp_body(g, carry):
        c0 = NBUF * g
        for b in range(NBUF):
            buf = bufs[b]
            pltpu.make_async_copy(
                emb_hbm.at[idx_v.at[0]], buf, sems[b]).wait()
            a = [buf[0, pl.ds(16 * j, 16)] for j in range(NJ)]
            c = [buf[HALF, pl.ds(16 * j, 16)] for j in range(NJ)]
            for t in range(1, HALF):
                for j in range(NJ):
                    a[j] = a[j] + buf[t, pl.ds(16 * j, 16)]
            for t in range(HALF + 1, CHUNK):
                for j in range(NJ):
                    c[j] = c[j] + buf[t, pl.ds(16 * j, 16)]
            r = (NBUF // 2) * g + b // 2

            @pl.when(g < NGRP - 1)
            def _():
                pltpu.async_copy(
                    emb_hbm.at[idx_v.at[c0 + b + NBUF]], buf, sems[b])

            if b % 2 == 0:
                for j in range(NJ):
                    acc_v[r, pl.ds(16 * j, 16)] = a[j] + c[j]
            else:
                for j in range(NJ):
                    acc_v[r, pl.ds(16 * j, 16)] = (
                        acc_v[r, pl.ds(16 * j, 16)] + (a[j] + c[j]))
        return carry

    lax.fori_loop(0, NGRP, grp_body, 0)
    pltpu.sync_copy(acc_v, out_hbm.at[pl.ds(wid * ROWS_PER_W, ROWS_PER_W)])


@functools.lru_cache(maxsize=1)
def _make_pool():
    return pl.kernel(
        _pool_body,
        out_type=jax.ShapeDtypeStruct((B, EMB_DIM), jnp.float32),
        mesh=plsc.VectorSubcoreMesh(core_axis_name="c", subcore_axis_name="s"),
        compiler_params=pltpu.CompilerParams(use_tc_tiling_on_sc=False),
        scratch_types=(
            [pltpu.VMEM((CHUNKS_PER_W, CHUNK), jnp.int32)]
            + [pltpu.VMEM((CHUNK, EMB_DIM), jnp.float32)
               for _ in range(NBUF)]
            + [pltpu.VMEM((ROWS_PER_W, EMB_DIM), jnp.float32)]
            + [pltpu.SemaphoreType.DMA for _ in range(NBUF)]
        ),
    )


def _mlp_body(x_ref, w1t_ref, b1_ref, w2t_ref, b2_ref, o_ref):
    x = x_ref[:] * (1.0 / L)
    h = jnp.dot(x, w1t_ref[:], preferred_element_type=jnp.float32) + b1_ref[:]
    h = jnp.maximum(h, 0.0)
    o = jnp.dot(h, w2t_ref[:], preferred_element_type=jnp.float32) + b2_ref[:]
    m = jnp.max(o, axis=1, keepdims=True)
    lse = jnp.log(jnp.sum(jnp.exp(o - m), axis=1, keepdims=True)) + m
    o_ref[:] = o - lse


def _mlp(sums, w1t, b1_2d, w2tp, b2p):
    blk = B // 4
    return pl.pallas_call(
        _mlp_body,
        grid=(4,),
        in_specs=[
            pl.BlockSpec((blk, EMB_DIM), lambda i: (i, 0)),
            pl.BlockSpec((EMB_DIM, HIDDEN), lambda i: (0, 0)),
            pl.BlockSpec((1, HIDDEN), lambda i: (0, 0)),
            pl.BlockSpec((HIDDEN, OUT_PAD), lambda i: (0, 0)),
            pl.BlockSpec((1, OUT_PAD), lambda i: (0, 0)),
        ],
        out_specs=pl.BlockSpec((blk, OUT_PAD), lambda i: (i, 0)),
        out_shape=jax.ShapeDtypeStruct((B, OUT_PAD), jnp.float32),
    )(sums, w1t, b1_2d, w2tp, b2p)


def kernel(subword_indices, emb, W1, b1, W2, b2):
    idx = subword_indices.astype(jnp.int32).reshape(B * L // CHUNK, CHUNK)
    sums = _make_pool()(idx, emb)
    w1t = W1.T
    b1_2d = b1.reshape(1, HIDDEN)
    w2tp = jnp.zeros((HIDDEN, OUT_PAD), jnp.float32).at[:, :OUT].set(W2.T)
    b2p = jnp.full((1, OUT_PAD), -1e30, jnp.float32).at[0, :OUT].set(b2)
    out = _mlp(sums, w1t, b1_2d, w2tp, b2p)
    return out[:, :OUT]
